# baseline (speedup 1.0000x reference)
import functools

import jax
import jax.numpy as jnp
from jax import lax
from jax.experimental import pallas as pl
from jax.experimental.pallas import tpu as pltpu
from jax.experimental.pallas import tpu_sc as plsc

_C = 1000
_CP = 1024
_B = 16384
_D = 512
_NC = 2
_NS = 16
_L = 16
_W = _D // (_NC * _NS)
_H = _D // _NC
_SC = 512
_NSC = _B // _SC
_SPT = _SC // _NS
_FB = 125
_NFB = _C // _FB


def _sc_stats(x, labs16, rm, rv, cc16):
    mesh = plsc.VectorSubcoreMesh(core_axis_name="c", subcore_axis_name="s")

    @functools.partial(
        pl.kernel,
        out_type=(
            jax.ShapeDtypeStruct((_C, _D), jnp.float32),
            jax.ShapeDtypeStruct((_C, _D), jnp.float32),
            jax.ShapeDtypeStruct((_CP, _L), jnp.float32),
        ),
        mesh=mesh,
        compiler_params=pltpu.CompilerParams(
            use_tc_tiling_on_sc=False, needs_layout_passes=False
        ),
        scratch_types=[
            pltpu.VMEM_SHARED((_SC, _H), jnp.float32),
            pltpu.VMEM_SHARED((_SC, _H), jnp.float32),
            pltpu.VMEM((_CP * _L,), jnp.float32),
            pltpu.VMEM((_CP * _L,), jnp.float32),
            pltpu.VMEM((_CP * _L,), jnp.float32),
            pltpu.VMEM((_SPT, _H), jnp.float32),
            pltpu.VMEM((_SPT, _H), jnp.float32),
            pltpu.VMEM((_SC,), jnp.int32),
            pltpu.VMEM((_SC, _W), jnp.float32),
            pltpu.SemaphoreType.DMA,
            pltpu.SemaphoreType.DMA,
            pltpu.VMEM((_FB, _L), jnp.float32),
            pltpu.VMEM((_FB, _L), jnp.float32),
            pltpu.VMEM((_FB, _L), jnp.float32),
            pltpu.VMEM((_FB, _L), jnp.float32),
            pltpu.VMEM((_FB, _L), jnp.float32),
            pltpu.VMEM((_FB, _L), jnp.float32),
        ],
    )
    def k(x_h, lab_h, rm_h, rv_h, cc_h, om_h, ov_h, oc_h,
          stg0, stg1, sum_t, ss_t, cnt_t, xw0, xw1, liv, xv, sw0, sw1,
          rm_b, rv_b, cc_b, om_b, ov_b, on_b):
        cid = lax.axis_index("c")
        sid = lax.axis_index("s")
        hb = cid * _H
        cb = hb + sid * _W

        def wide_cp(sc, buf, sem):
            return pltpu.make_async_copy(
                x_h.at[pl.ds(sc * _SC + sid * _SPT, _SPT), pl.ds(hb, _H)],
                buf,
                sem,
            )

        wide_cp(0, xw0, sw0).start()

        zero = jnp.zeros((_L,), jnp.float32)

        @plsc.parallel_loop(0, _CP, unroll=8)
        def _(i):
            sl = pl.ds(i * _L, _L)
            sum_t[sl] = zero
            ss_t[sl] = zero
            cnt_t[sl] = zero

        one = jnp.ones((_L,), jnp.float32)
        lanes = lax.iota(jnp.int32, _L)
        consts_r = [jnp.full((_L,), r, jnp.int32) for r in range(_L)]

        def consume():
            @plsc.parallel_loop(0, _SC // _L, unroll=2)
            def _(g):
                g0 = g * _L
                l16 = liv[pl.ds(g0, _L)]
                plsc.addupdate_scatter(cnt_t, [l16], one)
                gbase = jnp.full((_L,), g0, jnp.int32)
                for r in range(_L):
                    i = g * _L + r
                    a = plsc.load_gather(liv, [gbase + consts_r[r]])
                    addr = a + lanes
                    v = xv[i, pl.ds(0, _W)]
                    plsc.addupdate_scatter(sum_t, [addr], v)
                    plsc.addupdate_scatter(ss_t, [addr], v * v)

        def super_chunk(hc, _):
            for b, (xwb, swb, stg) in enumerate(
                ((xw0, sw0, stg0), (xw1, sw1, stg1))
            ):
                sc = 2 * hc + b
                wide_cp(sc, xwb, swb).wait()
                pltpu.sync_copy(xwb, stg.at[pl.ds(sid * _SPT, _SPT)])

                @pl.when(sc + 1 < _NSC)
                def _():
                    nxt = (xw1, sw1) if b == 0 else (xw0, sw0)
                    wide_cp(sc + 1, nxt[0], nxt[1]).start()

                pltpu.sync_copy(lab_h.at[pl.ds(sc * _SC, _SC)], liv)
                plsc.subcore_barrier()
                pltpu.sync_copy(
                    stg.at[pl.ds(0, _SC), pl.ds(sid * _W, _W)], xv
                )
                consume()
            return 0

        lax.fori_loop(0, _NSC // 2, super_chunk, 0)

        def blk(b, _):
            r0 = b * _FB
            pltpu.sync_copy(rm_h.at[pl.ds(r0, _FB), pl.ds(cb, _W)], rm_b)
            pltpu.sync_copy(rv_h.at[pl.ds(r0, _FB), pl.ds(cb, _W)], rv_b)
            pltpu.sync_copy(cc_h.at[pl.ds(r0, _FB)], cc_b)

            @plsc.parallel_loop(0, _FB, unroll=5)
            def _(i):
                r = r0 + i
                sl = pl.ds(r * _L, _L)
                nb = plsc.load_gather(cnt_t, [jnp.full((_L,), r, jnp.int32) * _L])
                na = cc_b[i, pl.ds(0, _L)]
                n = na + nb
                on_b[i, pl.ds(0, _L)] = n
                rn = 1.0 / jnp.maximum(n, 1.0)
                s_ = sum_t[sl]
                q_ = ss_t[sl]
                m_ = rm_b[i, pl.ds(0, _L)]
                v_ = rv_b[i, pl.ds(0, _L)]
                mean = (na * m_ + s_) * rn
                om_b[i, pl.ds(0, _L)] = mean
                ov_b[i, pl.ds(0, _L)] = (na * (v_ + m_ * m_) + q_) * rn - mean * mean

            pltpu.sync_copy(om_b, om_h.at[pl.ds(r0, _FB), pl.ds(cb, _W)])
            pltpu.sync_copy(ov_b, ov_h.at[pl.ds(r0, _FB), pl.ds(cb, _W)])

            @pl.when(jnp.logical_and(cid == 0, sid == 0))
            def _():
                pltpu.sync_copy(on_b, oc_h.at[pl.ds(r0, _FB)])

            return 0

        lax.fori_loop(0, _NFB, blk, 0)

    return k(x, labs16, rm, rv, cc16)


def kernel(x, labels, running_mean, running_var, class_count):
    cc16 = jnp.pad(
        jnp.broadcast_to(class_count, (_C, _L)), ((0, _CP - _C), (0, 0))
    )
    labs16 = labels.astype(jnp.int32) * _L
    um, uv, cn = _sc_stats(x, labs16, running_mean, running_var, cc16)
    return um, uv, cn[:_C, :1]


# speedup vs baseline: 1.2776x; 1.2776x over previous
import functools

import jax
import jax.numpy as jnp
from jax import lax
from jax.experimental import pallas as pl
from jax.experimental.pallas import tpu as pltpu
from jax.experimental.pallas import tpu_sc as plsc

_C = 1000
_CP = 1024
_B = 16384
_D = 512
_NC = 2
_NS = 16
_L = 16
_W = _D // (_NC * _NS)
_CH = 256
_NCH = _B // _CH
_FB = 125
_NFB = _C // _FB


def _sc_stats(x, labels, rm, rv, cc16):
    mesh = plsc.VectorSubcoreMesh(core_axis_name="c", subcore_axis_name="s")

    @functools.partial(
        pl.kernel,
        out_type=(
            jax.ShapeDtypeStruct((_C, _D), jnp.float32),
            jax.ShapeDtypeStruct((_C, _D), jnp.float32),
            jax.ShapeDtypeStruct((_CP, _L), jnp.float32),
        ),
        mesh=mesh,
        compiler_params=pltpu.CompilerParams(
            use_tc_tiling_on_sc=False, needs_layout_passes=False
        ),
        scratch_types=[
            pltpu.VMEM((_CP * _L,), jnp.float32),
            pltpu.VMEM((_CP * _L,), jnp.float32),
            pltpu.VMEM((_CP * _L,), jnp.float32),
            pltpu.VMEM((_CH, _W), jnp.float32),
            pltpu.VMEM((_CH, _W), jnp.float32),
            pltpu.VMEM((_CH, _W), jnp.float32),
            pltpu.VMEM((_CH, _W), jnp.float32),
            pltpu.VMEM((_CH,), jnp.int32),
            pltpu.VMEM((_CH,), jnp.int32),
            pltpu.VMEM((_CH,), jnp.int32),
            pltpu.VMEM((_CH,), jnp.int32),
            pltpu.SemaphoreType.DMA,
            pltpu.SemaphoreType.DMA,
            pltpu.SemaphoreType.DMA,
            pltpu.SemaphoreType.DMA,
            pltpu.SemaphoreType.DMA,
            pltpu.SemaphoreType.DMA,
            pltpu.SemaphoreType.DMA,
            pltpu.SemaphoreType.DMA,
            pltpu.VMEM((_FB, _L), jnp.float32),
            pltpu.VMEM((_FB, _L), jnp.float32),
            pltpu.VMEM((_FB, _L), jnp.float32),
            pltpu.VMEM((_FB, _L), jnp.float32),
            pltpu.VMEM((_FB, _L), jnp.float32),
            pltpu.VMEM((_FB, _L), jnp.float32),
        ],
    )
    def k(x_h, lab_h, rm_h, rv_h, cc_h, om_h, ov_h, oc_h,
          sum_t, ss_t, cnt_t, xv0, xv1, xv2, xv3, iv0, iv1, iv2, iv3,
          sx0, sx1, sx2, sx3, si0, si1, si2, si3,
          rm_b, rv_b, cc_b, om_b, ov_b, on_b):
        cid = lax.axis_index("c")
        sid = lax.axis_index("s")
        w = cid * _NS + sid
        cb = w * _W

        def x_cp(ci, buf, sem):
            return pltpu.make_async_copy(
                x_h.at[pl.ds(ci * _CH, _CH), pl.ds(cb, _W)], buf, sem
            )

        def l_cp(ci, buf, sem):
            return pltpu.make_async_copy(lab_h.at[pl.ds(ci * _CH, _CH)], buf, sem)

        bufs = (
            (xv0, iv0, sx0, si0),
            (xv1, iv1, sx1, si1),
            (xv2, iv2, sx2, si2),
            (xv3, iv3, sx3, si3),
        )
        for ci in range(4):
            xvb, ivb, sxb, sib = bufs[ci]
            x_cp(ci, xvb, sxb).start()
            l_cp(ci, ivb, sib).start()

        zero = jnp.zeros((_L,), jnp.float32)

        @plsc.parallel_loop(0, _CP, unroll=8)
        def _(i):
            sl = pl.ds(i * _L, _L)
            sum_t[sl] = zero
            ss_t[sl] = zero
            cnt_t[sl] = zero

        one = jnp.ones((_L,), jnp.float32)
        lanes = lax.iota(jnp.int32, _L)

        consts_r = [jnp.full((_L,), r, jnp.int32) for r in range(_L)]

        def consume(ci, xvb, ivb):
            @plsc.parallel_loop(0, _CH // _L, unroll=2)
            def _(g):
                l16 = ivb[pl.ds(g * _L, _L)]
                plsc.addupdate_scatter(cnt_t, [l16], one)
                gbase = jnp.full((_L,), g * _L, jnp.int32)
                for r in range(_L):
                    i = g * _L + r
                    a = plsc.load_gather(ivb, [gbase + consts_r[r]])
                    addr = a + lanes
                    v = xvb[i, pl.ds(0, _W)]
                    plsc.addupdate_scatter(sum_t, [addr], v)
                    plsc.addupdate_scatter(ss_t, [addr], v * v)

        def outer(cc, _):
            for b, (xvb, ivb, sxb, sib) in enumerate(bufs):
                ci = 4 * cc + b
                x_cp(ci, xvb, sxb).wait()
                l_cp(ci, ivb, sib).wait()
                consume(ci, xvb, ivb)

                @pl.when(ci + 4 < _NCH)
                def _():
                    x_cp(ci + 4, xvb, sxb).start()
                    l_cp(ci + 4, ivb, sib).start()

            return 0

        lax.fori_loop(0, _NCH // 4, outer, 0)

        def blk(b, _):
            r0 = b * _FB
            pltpu.sync_copy(rm_h.at[pl.ds(r0, _FB), pl.ds(cb, _W)], rm_b)
            pltpu.sync_copy(rv_h.at[pl.ds(r0, _FB), pl.ds(cb, _W)], rv_b)
            pltpu.sync_copy(cc_h.at[pl.ds(r0, _FB)], cc_b)

            @plsc.parallel_loop(0, _FB, unroll=5)
            def _(i):
                r = r0 + i
                sl = pl.ds(r * _L, _L)
                nb = plsc.load_gather(cnt_t, [jnp.full((_L,), r, jnp.int32) * _L])
                na = cc_b[i, pl.ds(0, _L)]
                n = na + nb
                on_b[i, pl.ds(0, _L)] = n
                rn = 1.0 / jnp.maximum(n, 1.0)
                s_ = sum_t[sl]
                q_ = ss_t[sl]
                m_ = rm_b[i, pl.ds(0, _L)]
                v_ = rv_b[i, pl.ds(0, _L)]
                mean = (na * m_ + s_) * rn
                om_b[i, pl.ds(0, _L)] = mean
                ov_b[i, pl.ds(0, _L)] = (na * (v_ + m_ * m_) + q_) * rn - mean * mean
            pltpu.sync_copy(om_b, om_h.at[pl.ds(r0, _FB), pl.ds(cb, _W)])
            pltpu.sync_copy(ov_b, ov_h.at[pl.ds(r0, _FB), pl.ds(cb, _W)])

            @pl.when(w == 0)
            def _():
                pltpu.sync_copy(on_b, oc_h.at[pl.ds(r0, _FB)])

            return 0

        lax.fori_loop(0, _NFB, blk, 0)

    return k(x, labels, rm, rv, cc16)


def kernel(x, labels, running_mean, running_var, class_count):
    cc16 = jnp.pad(
        jnp.broadcast_to(class_count, (_C, _L)), ((0, _CP - _C), (0, 0))
    )
    labs16 = labels.astype(jnp.int32) * _L
    um, uv, cn = _sc_stats(x, labs16, running_mean, running_var, cc16)
    return um, uv, cn[:_C, :1]


# speedup vs baseline: 1.4022x; 1.0976x over previous
import functools

import jax
import jax.numpy as jnp
from jax import lax
from jax.experimental import pallas as pl
from jax.experimental.pallas import tpu as pltpu
from jax.experimental.pallas import tpu_sc as plsc

_C = 1000
_CP = 1024
_B = 16384
_D = 512
_NC = 2
_NS = 16
_L = 16
_W = _D // (_NC * _NS)
_CH = 256
_NCH = _B // _CH
_FB = 125
_NFB = _C // _FB


def _sc_stats(x, labels, rm, rv, cc16):
    mesh = plsc.VectorSubcoreMesh(core_axis_name="c", subcore_axis_name="s")

    @functools.partial(
        pl.kernel,
        out_type=(
            jax.ShapeDtypeStruct((_C, _D), jnp.float32),
            jax.ShapeDtypeStruct((_C, _D), jnp.float32),
            jax.ShapeDtypeStruct((_CP, _L), jnp.float32),
        ),
        mesh=mesh,
        compiler_params=pltpu.CompilerParams(
            use_tc_tiling_on_sc=False, needs_layout_passes=False
        ),
        scratch_types=[
            pltpu.VMEM((_CP * _L,), jnp.float32),
            pltpu.VMEM((_CP * _L,), jnp.float32),
            pltpu.VMEM((_CP * _L,), jnp.float32),
            pltpu.VMEM((_CH, _W), jnp.float32),
            pltpu.VMEM((_CH, _W), jnp.float32),
            pltpu.VMEM((_CH,), jnp.int32),
            pltpu.VMEM((_CH,), jnp.int32),
            pltpu.SemaphoreType.DMA,
            pltpu.SemaphoreType.DMA,
            pltpu.SemaphoreType.DMA,
            pltpu.SemaphoreType.DMA,
            pltpu.VMEM((_FB, _L), jnp.float32),
            pltpu.VMEM((_FB, _L), jnp.float32),
            pltpu.VMEM((_FB, _L), jnp.float32),
            pltpu.VMEM((_FB, _L), jnp.float32),
            pltpu.VMEM((_FB, _L), jnp.float32),
            pltpu.VMEM((_FB, _L), jnp.float32),
        ],
    )
    def k(x_h, lab_h, rm_h, rv_h, cc_h, om_h, ov_h, oc_h,
          sum_t, ss_t, cnt_t, xv0, xv1, iv0, iv1, sx0, sx1, si0, si1,
          rm_b, rv_b, cc_b, om_b, ov_b, on_b):
        cid = lax.axis_index("c")
        sid = lax.axis_index("s")
        w = cid * _NS + sid
        cb = w * _W

        def x_cp(ci, buf, sem):
            return pltpu.make_async_copy(
                x_h.at[pl.ds(ci * _CH, _CH), pl.ds(cb, _W)], buf, sem
            )

        def l_cp(ci, buf, sem):
            return pltpu.make_async_copy(lab_h.at[pl.ds(ci * _CH, _CH)], buf, sem)

        x_cp(0, xv0, sx0).start()
        l_cp(0, iv0, si0).start()
        x_cp(1, xv1, sx1).start()
        l_cp(1, iv1, si1).start()

        zero = jnp.zeros((_L,), jnp.float32)

        @plsc.parallel_loop(0, _CP, unroll=8)
        def _(i):
            sl = pl.ds(i * _L, _L)
            sum_t[sl] = zero
            ss_t[sl] = zero
            cnt_t[sl] = zero

        one = jnp.ones((_L,), jnp.float32)
        lanes = lax.iota(jnp.int32, _L)

        def consume(ci, xvb, ivb):
            @plsc.parallel_loop(0, _CH, unroll=16)
            def _(i):
                lab = plsc.load_gather(ivb, [jnp.full((_L,), i, jnp.int32)])
                addr = lab * _L + lanes
                v = xvb[i, pl.ds(0, _W)]
                plsc.addupdate_scatter(sum_t, [addr], v)
                plsc.addupdate_scatter(ss_t, [addr], v * v)
                plsc.addupdate_scatter(cnt_t, [addr], one)

        def outer(cc, _):
            for b, (xvb, ivb, sxb, sib) in enumerate(
                ((xv0, iv0, sx0, si0), (xv1, iv1, sx1, si1))
            ):
                ci = 2 * cc + b
                x_cp(ci, xvb, sxb).wait()
                l_cp(ci, ivb, sib).wait()
                consume(ci, xvb, ivb)

                @pl.when(ci + 2 < _NCH)
                def _():
                    x_cp(ci + 2, xvb, sxb).start()
                    l_cp(ci + 2, ivb, sib).start()

            return 0

        lax.fori_loop(0, _NCH // 2, outer, 0)

        def blk(b, _):
            r0 = b * _FB
            pltpu.sync_copy(rm_h.at[pl.ds(r0, _FB), pl.ds(cb, _W)], rm_b)
            pltpu.sync_copy(rv_h.at[pl.ds(r0, _FB), pl.ds(cb, _W)], rv_b)
            pltpu.sync_copy(cc_h.at[pl.ds(r0, _FB)], cc_b)

            @plsc.parallel_loop(0, _FB, unroll=5)
            def _(i):
                r = r0 + i
                sl = pl.ds(r * _L, _L)
                nb = cnt_t[sl]
                na = cc_b[i, pl.ds(0, _L)]
                n = na + nb
                on_b[i, pl.ds(0, _L)] = n
                rn = 1.0 / jnp.maximum(n, 1.0)
                s_ = sum_t[sl]
                q_ = ss_t[sl]
                m_ = rm_b[i, pl.ds(0, _L)]
                v_ = rv_b[i, pl.ds(0, _L)]
                mean = (na * m_ + s_) * rn
                om_b[i, pl.ds(0, _L)] = mean
                ov_b[i, pl.ds(0, _L)] = (na * (v_ + m_ * m_) + q_) * rn - mean * mean
            pltpu.sync_copy(om_b, om_h.at[pl.ds(r0, _FB), pl.ds(cb, _W)])
            pltpu.sync_copy(ov_b, ov_h.at[pl.ds(r0, _FB), pl.ds(cb, _W)])

            @pl.when(w == 0)
            def _():
                pltpu.sync_copy(on_b, oc_h.at[pl.ds(r0, _FB)])

            return 0

        lax.fori_loop(0, _NFB, blk, 0)

    return k(x, labels, rm, rv, cc16)


def kernel(x, labels, running_mean, running_var, class_count):
    cc16 = jnp.pad(
        jnp.broadcast_to(class_count, (_C, _L)), ((0, _CP - _C), (0, 0))
    )
    um, uv, cn = _sc_stats(x, labels, running_mean, running_var, cc16)
    return um, uv, cn[:_C, :1]
